# baseline (speedup 1.0000x reference)
import jax
import jax.numpy as jnp
from jax.experimental import pallas as pl
from jax.experimental.pallas import tpu as pltpu


_SRC_ROWS = (0, 1, 2, 3, 7, 8, 9, 10)
_CHUNKS = 4


def _gather_copy_kernel(x_ref, o_ref, sem):
    copies = []
    step = 2048 // _CHUNKS
    for i, r in enumerate(_SRC_ROWS):
        for c in range(_CHUNKS):
            copies.append(pltpu.make_async_copy(
                x_ref.at[r, pl.ds(c * step, step), :],
                o_ref.at[i, pl.ds(c * step, step), :],
                sem.at[i * _CHUNKS + c]))
    for cp in copies:
        cp.start()
    for cp in copies:
        cp.wait()


def kernel(x):
    xf = x.reshape(14, 2048, 2048)
    return pl.pallas_call(
        _gather_copy_kernel,
        out_shape=jax.ShapeDtypeStruct((8, 2048, 2048), x.dtype),
        in_specs=[pl.BlockSpec(memory_space=pltpu.MemorySpace.HBM)],
        out_specs=pl.BlockSpec(memory_space=pltpu.MemorySpace.HBM),
        scratch_shapes=[pltpu.SemaphoreType.DMA((len(_SRC_ROWS) * _CHUNKS,))],
    )(xf)


# speedup vs baseline: 47.9577x; 47.9577x over previous
import jax
import jax.numpy as jnp
from jax.experimental import pallas as pl
from jax.experimental.pallas import tpu as pltpu


_BLK = 512


def _gather_copy_kernel(x_ref, o_ref):
    o_ref[...] = x_ref[...]


def kernel(x):
    xf = x.reshape(14, 2048, 2048)
    return pl.pallas_call(
        _gather_copy_kernel,
        out_shape=jax.ShapeDtypeStruct((8, 2048, 2048), x.dtype),
        grid=(8, 2048 // _BLK),
        in_specs=[pl.BlockSpec((1, _BLK, 2048),
                               lambda i, j: (i + 3 * (i // 4), j, 0))],
        out_specs=pl.BlockSpec((1, _BLK, 2048), lambda i, j: (i, j, 0)),
    )(xf)


# speedup vs baseline: 48.8092x; 1.0178x over previous
import jax
import jax.numpy as jnp
from jax.experimental import pallas as pl
from jax.experimental.pallas import tpu as pltpu


_BLK = 1024


def _gather_copy_kernel(x_ref, o_ref):
    o_ref[...] = x_ref[...]


def kernel(x):
    xf = x.reshape(14, 2048, 2048)
    return pl.pallas_call(
        _gather_copy_kernel,
        out_shape=jax.ShapeDtypeStruct((8, 2048, 2048), x.dtype),
        grid=(8, 2048 // _BLK),
        in_specs=[pl.BlockSpec((1, _BLK, 2048),
                               lambda i, j: (i + 3 * (i // 4), j, 0))],
        out_specs=pl.BlockSpec((1, _BLK, 2048), lambda i, j: (i, j, 0)),
    )(xf)
